# baseline (speedup 1.0000x reference)
import jax
import jax.numpy as jnp
from jax.experimental import pallas as pl

EPS = 1e-6
NEG_SLOPE = 0.2
K = 20
DIMS = [1, 21, 21, 42, 85, 341]


def _knn(x, k):
    inner = -2.0 * jnp.einsum('bcn,bcm->bnm', x, x)
    xx = jnp.sum(x ** 2, axis=1, keepdims=True)
    pairwise = -xx - inner - jnp.transpose(xx, (0, 2, 1))
    _, idx = jax.lax.top_k(pairwise, k)
    return idx


def _get_graph_feature(x, k):
    B = x.shape[0]
    N = x.shape[3]
    xf = x.reshape(B, -1, N)
    idx = _knn(xf, k)
    num_dims = xf.shape[1] // 3
    xt = jnp.transpose(xf, (0, 2, 1))
    feature = jax.vmap(lambda xb, ib: xb[ib])(xt, idx)
    feature = feature.reshape(B, N, k, num_dims, 3)
    xr = xt.reshape(B, N, 1, num_dims, 3)
    xr = jnp.broadcast_to(xr, (B, N, k, num_dims, 3))
    feature = jnp.concatenate([feature - xr, xr], axis=3)
    return jnp.transpose(feature, (0, 3, 4, 1, 2))


def _vn_linear_leakyrelu(x, w_feat, w_dir):
    p = jnp.einsum('oc,bc...->bo...', w_feat, x)
    d = jnp.einsum('oc,bc...->bo...', w_dir, x)
    dot = jnp.sum(p * d, axis=2, keepdims=True)
    mask = (dot >= 0).astype(x.dtype)
    d_norm_sq = jnp.sum(d * d, axis=2, keepdims=True)
    return NEG_SLOPE * p + (1.0 - NEG_SLOPE) * (mask * p + (1.0 - mask) * (p - (dot / (d_norm_sq + EPS)) * d))


def _final_kernel(xc_ref, wf_ref, wd_ref, out_ref):
    xc = xc_ref[0]
    p = jnp.dot(wf_ref[...], xc, preferred_element_type=jnp.float32)
    d = jnp.dot(wd_ref[...], xc, preferred_element_type=jnp.float32)
    N = xc.shape[1] // 3
    p3 = p.reshape(341, 3, N)
    d3 = d.reshape(1, 3, N)
    dot = jnp.sum(p3 * d3, axis=1, keepdims=True)
    dnsq = jnp.sum(d3 * d3, axis=1, keepdims=True)
    neg = (dot < 0).astype(jnp.float32)
    out = p3 - (1.0 - NEG_SLOPE) * neg * (dot / (dnsq + EPS)) * d3
    out_ref[0] = jnp.mean(out, axis=2)


def kernel(x, w_feat_0, w_dir_0, w_feat_1, w_dir_1, w_feat_2, w_dir_2,
           w_feat_3, w_dir_3, w_feat_4, w_dir_4):
    params = {
        "w_feat_0": w_feat_0, "w_dir_0": w_dir_0,
        "w_feat_1": w_feat_1, "w_dir_1": w_dir_1,
        "w_feat_2": w_feat_2, "w_dir_2": w_dir_2,
        "w_feat_3": w_feat_3, "w_dir_3": w_dir_3,
    }
    xx = x[:, None]
    x_list = []
    for i in range(len(DIMS) - 2):
        f = _get_graph_feature(xx, K)
        h = _vn_linear_leakyrelu(f, params["w_feat_%d" % i], params["w_dir_%d" % i])
        xx = jnp.mean(h, axis=-1)
        x_list.append(xx)
    xc = jnp.concatenate(x_list, axis=1)
    B, C, _, N = xc.shape
    xc2 = xc.reshape(B, C, 3 * N)
    out = pl.pallas_call(
        _final_kernel,
        out_shape=jax.ShapeDtypeStruct((B, 341, 3), jnp.float32),
        grid=(B,),
        in_specs=[
            pl.BlockSpec((1, C, 3 * N), lambda b: (b, 0, 0)),
            pl.BlockSpec((341, C), lambda b: (0, 0)),
            pl.BlockSpec((1, C), lambda b: (0, 0)),
        ],
        out_specs=pl.BlockSpec((1, 341, 3), lambda b: (b, 0, 0)),
    )(xc2, w_feat_4, w_dir_4)
    return out


# speedup vs baseline: 4.1736x; 4.1736x over previous
import functools

import jax
import jax.numpy as jnp
from jax import lax
from jax.experimental import pallas as pl
from jax.experimental.pallas import tpu as pltpu
from jax.experimental.pallas import tpu_sc as plsc

EPS = 1e-6
NEG_SLOPE = 0.2
K = 20
N = 1024
B = 4
BN = B * N
NW = 32
PPW = BN // NW
PC = 8
BLK = 128
HI = lax.Precision.HIGHEST

LAYERS = [
    (1, 16 // 3 + 1, 21, 32),
    (21, 32, 21, 32),
    (21, 32, 42, 48),
    (42, 48, 85, 96),
]
C3P = [16, 96, 96, 144]


def _padk(w_t, cp_in, copm):
    cin, cout = w_t.shape
    wp = jnp.zeros((cp_in, copm), jnp.float32).at[:cin, :cout].set(w_t)
    return jnp.kron(jnp.eye(3, dtype=jnp.float32), wp)


def _pad_row(w_t, c3p, copm):
    cin, cout = w_t.shape
    k3 = jnp.kron(jnp.eye(3, dtype=jnp.float32), w_t)
    out = jnp.zeros((c3p, 3 * copm), jnp.float32)
    col = jnp.zeros((3 * copm,), jnp.float32)
    full = jnp.zeros((3, 3 * copm), jnp.float32)
    for d in range(3):
        full = full.at[:, d * copm:d * copm + cout].set(k3[:, d * cout:(d + 1) * cout])
    del col
    return out.at[:3, :].set(full)


def _tc_knn_kernel(x_ref, idx_ref):
    b = pl.program_id(0)
    x = x_ref[...]
    xb = x.astype(jnp.bfloat16)
    g = lax.dot_general(xb, xb, (((1,), (1,)), ((), ())),
                        preferred_element_type=jnp.float32)
    xx2 = x * x
    sq_col = jnp.sum(xx2, axis=1, keepdims=True)
    ones = jnp.ones((1, x.shape[1]), jnp.float32)
    sq_row = lax.dot_general(ones, xx2, (((1,), (1,)), ((), ())),
                             precision=HI, preferred_element_type=jnp.float32)
    pairwise = (-sq_col - (-2.0 * g)) - sq_row

    ci = lax.broadcasted_iota(jnp.int32, (N, N), 1)
    cik = lax.broadcasted_iota(jnp.int32, (N, K), 1)
    base = b * N
    idxacc = lax.broadcasted_iota(jnp.int32, (N, K), 0) + base
    work = pairwise
    for t in range(K):
        m = jnp.max(work, axis=1, keepdims=True)
        sel = jnp.where(work == m, ci, N)
        idx_t = jnp.min(sel, axis=1, keepdims=True)
        idxacc = jnp.where(cik == t, idx_t + base, idxacc)
        work = jnp.where(ci == idx_t, -jnp.inf, work)
    idx_ref[...] = idxacc


def _tc_knn(x2, c3p):
    return pl.pallas_call(
        _tc_knn_kernel,
        grid=(B,),
        in_specs=[pl.BlockSpec((N, c3p), lambda b: (b, 0))],
        out_specs=pl.BlockSpec((N, K), lambda b: (b, 0)),
        out_shape=jax.ShapeDtypeStruct((BN, K), jnp.int32),
    )(x2)


def _sc_gather_body(idx_hbm, xtab_hbm, out_hbm, idx_v, gbuf, gsem):
    cid = lax.axis_index("c")
    sid = lax.axis_index("s")
    wid = sid * 2 + cid
    ebase = wid * (PPW * K)

    pltpu.sync_copy(idx_hbm.at[pl.ds(ebase, PPW * K)], idx_v)

    nchunk = PPW // PC
    ec = PC * K

    def chunk(i, _):
        pltpu.async_copy(
            xtab_hbm.at[idx_v.at[pl.ds(i * ec, ec)]], gbuf, gsem).wait()
        pltpu.sync_copy(gbuf, out_hbm.at[pl.ds(ebase + i * ec, ec)])
        return ()

    lax.fori_loop(0, nchunk, chunk, ())


def _sc_gather(idx, xtab, c3p):
    mesh = plsc.VectorSubcoreMesh(core_axis_name="c", subcore_axis_name="s")
    kern = pl.kernel(
        _sc_gather_body,
        out_type=jax.ShapeDtypeStruct((BN * K, c3p), jnp.float32),
        mesh=mesh,
        compiler_params=pltpu.CompilerParams(use_tc_tiling_on_sc=False),
        scratch_types=[
            pltpu.VMEM((PPW * K,), jnp.int32),
            pltpu.VMEM((PC * K, c3p), jnp.float32),
            pltpu.SemaphoreType.DMA,
        ],
    )
    return kern(idx.reshape(BN * K), xtab)


def _tc_conv_kernel(coutp, g_ref, x_ref, wd_ref, wc_ref, out_ref):
    w3 = 3 * coutp
    gath = g_ref[...]
    xc = x_ref[...]
    c3p = xc.shape[1]
    g3 = gath.reshape(BLK, K, c3p)
    diff = g3 - xc[:, None, :]
    diffb = diff.reshape(BLK * K, c3p).astype(jnp.bfloat16)
    xb = xc.astype(jnp.bfloat16)
    pd1 = jnp.dot(diffb, wd_ref[...], preferred_element_type=jnp.float32)
    pd2 = jnp.dot(xb, wc_ref[...], preferred_element_type=jnp.float32)
    h = pd1.reshape(BLK, K, 6 * coutp) + pd2[:, None, :]
    px, py, pz = h[:, :, 0:coutp], h[:, :, coutp:2 * coutp], h[:, :, 2 * coutp:w3]
    dx, dy, dz = (h[:, :, w3:w3 + coutp], h[:, :, w3 + coutp:w3 + 2 * coutp],
                  h[:, :, w3 + 2 * coutp:2 * w3])
    dot = px * dx + py * dy + pz * dz
    dnsq = dx * dx + dy * dy + dz * dz
    mask = (dot >= 0).astype(jnp.float32)
    coef = (1.0 - mask) * (dot / (dnsq + EPS))
    ox = NEG_SLOPE * px + (1.0 - NEG_SLOPE) * (mask * px + (1.0 - mask) * px - coef * dx)
    oy = NEG_SLOPE * py + (1.0 - NEG_SLOPE) * (mask * py + (1.0 - mask) * py - coef * dy)
    oz = NEG_SLOPE * pz + (1.0 - NEG_SLOPE) * (mask * pz + (1.0 - mask) * pz - coef * dz)
    out_ref[...] = jnp.concatenate(
        [jnp.sum(ox, axis=1) / K, jnp.sum(oy, axis=1) / K, jnp.sum(oz, axis=1) / K],
        axis=1)


def _tc_conv(gmat, x2, wd_b, wc_b, coutp):
    c3p = x2.shape[1]
    w6 = 6 * coutp
    nblk = BN // BLK
    return pl.pallas_call(
        functools.partial(_tc_conv_kernel, coutp),
        grid=(nblk,),
        in_specs=[
            pl.BlockSpec((BLK * K, c3p), lambda i: (i, 0)),
            pl.BlockSpec((BLK, c3p), lambda i: (i, 0)),
            pl.BlockSpec((c3p, w6), lambda i: (0, 0)),
            pl.BlockSpec((c3p, w6), lambda i: (0, 0)),
        ],
        out_specs=pl.BlockSpec((BLK, 3 * coutp), lambda i: (i, 0)),
        out_shape=jax.ShapeDtypeStruct((BN, 3 * coutp), jnp.float32),
    )(gmat, x2, wd_b, wc_b)


def _tc_final_kernel(x0_ref, x1_ref, x2_ref, x3_ref,
                     wf0_ref, wf1_ref, wf2_ref, wf3_ref,
                     wd0_ref, wd1_ref, wd2_ref, wd3_ref, out_ref):
    cout = 341
    p = jnp.dot(x0_ref[...].astype(jnp.bfloat16), wf0_ref[...], preferred_element_type=jnp.float32)
    p += jnp.dot(x1_ref[...].astype(jnp.bfloat16), wf1_ref[...], preferred_element_type=jnp.float32)
    p += jnp.dot(x2_ref[...].astype(jnp.bfloat16), wf2_ref[...], preferred_element_type=jnp.float32)
    p += jnp.dot(x3_ref[...].astype(jnp.bfloat16), wf3_ref[...], preferred_element_type=jnp.float32)
    dv = jnp.dot(x0_ref[...].astype(jnp.bfloat16), wd0_ref[...], preferred_element_type=jnp.float32)
    dv += jnp.dot(x1_ref[...].astype(jnp.bfloat16), wd1_ref[...], preferred_element_type=jnp.float32)
    dv += jnp.dot(x2_ref[...].astype(jnp.bfloat16), wd2_ref[...], preferred_element_type=jnp.float32)
    dv += jnp.dot(x3_ref[...].astype(jnp.bfloat16), wd3_ref[...], preferred_element_type=jnp.float32)
    px, py, pz = p[:, 0:cout], p[:, cout:2 * cout], p[:, 2 * cout:3 * cout]
    dx, dy, dz = dv[:, 0:1], dv[:, 1:2], dv[:, 2:3]
    dot = px * dx + py * dy + pz * dz
    dnsq = dx * dx + dy * dy + dz * dz
    mask = (dot >= 0).astype(jnp.float32)
    coef = (1.0 - mask) * (dot / (dnsq + EPS))
    ox = NEG_SLOPE * px + (1.0 - NEG_SLOPE) * (mask * px + (1.0 - mask) * px - coef * dx)
    oy = NEG_SLOPE * py + (1.0 - NEG_SLOPE) * (mask * py + (1.0 - mask) * py - coef * dy)
    oz = NEG_SLOPE * pz + (1.0 - NEG_SLOPE) * (mask * pz + (1.0 - mask) * pz - coef * dz)
    out_ref[0] = jnp.concatenate(
        [jnp.sum(ox, axis=0, keepdims=True) / N,
         jnp.sum(oy, axis=0, keepdims=True) / N,
         jnp.sum(oz, axis=0, keepdims=True) / N], axis=0)


def _tc_final(xs, wf_list, wd_list):
    in_specs = []
    args = []
    for x in xs:
        f = x.shape[1]
        in_specs.append(pl.BlockSpec((N, f), lambda b: (b, 0)))
        args.append(x)
    for w in wf_list + wd_list:
        in_specs.append(pl.BlockSpec(w.shape, lambda b: (0, 0)))
        args.append(w)
    return pl.pallas_call(
        _tc_final_kernel,
        grid=(B,),
        in_specs=in_specs,
        out_specs=pl.BlockSpec((1, 3, 341), lambda b: (b, 0, 0)),
        out_shape=jax.ShapeDtypeStruct((B, 3, 341), jnp.float32),
    )(*args)


def kernel(x, w_feat_0, w_dir_0, w_feat_1, w_dir_1, w_feat_2, w_dir_2,
           w_feat_3, w_dir_3, w_feat_4, w_dir_4):
    wfs = [w_feat_0, w_feat_1, w_feat_2, w_feat_3]
    wds = [w_dir_0, w_dir_1, w_dir_2, w_dir_3]

    x2 = jnp.transpose(x, (0, 2, 1)).reshape(BN, 3)
    x2 = jnp.pad(x2, ((0, 0), (0, C3P[0] - 3)))

    xs = []
    for li, (cin, _, cout, coutp) in enumerate(LAYERS):
        c3p = C3P[li]
        wf, wd = wfs[li], wds[li]
        if li == 0:
            wdk = jnp.concatenate(
                [_pad_row(wf[:, :cin].T, c3p, coutp),
                 _pad_row(wd[:, :cin].T, c3p, coutp)], axis=1)
            wck = jnp.concatenate(
                [_pad_row(wf[:, cin:].T, c3p, coutp),
                 _pad_row(wd[:, cin:].T, c3p, coutp)], axis=1)
        else:
            cp_in = c3p // 3
            wdk = jnp.concatenate(
                [_padk(wf[:, :cin].T, cp_in, coutp),
                 _padk(wd[:, :cin].T, cp_in, coutp)], axis=1)
            wck = jnp.concatenate(
                [_padk(wf[:, cin:].T, cp_in, coutp),
                 _padk(wd[:, cin:].T, cp_in, coutp)], axis=1)
        idx = _tc_knn(x2, c3p)
        gmat = _sc_gather(idx, x2, c3p)
        x2 = _tc_conv(gmat, x2, wdk.astype(jnp.bfloat16),
                      wck.astype(jnp.bfloat16), coutp)
        xs.append(x2)

    wf4_list, wd4_list = [], []
    off = 0
    for (cin, _, cout, coutp) in LAYERS:
        wf4_list.append(_padk(w_feat_4[:, off:off + cout].T, coutp, 341).astype(jnp.bfloat16))
        wd4_list.append(_padk(w_dir_4[:, off:off + cout].T, coutp, 1).astype(jnp.bfloat16))
        off += cout
    out = _tc_final(xs, wf4_list, wd4_list)
    return jnp.transpose(out, (0, 2, 1))


# speedup vs baseline: 5.0709x; 1.2150x over previous
import functools

import jax
import jax.numpy as jnp
from jax import lax
from jax.experimental import pallas as pl
from jax.experimental.pallas import tpu as pltpu
from jax.experimental.pallas import tpu_sc as plsc

EPS = 1e-6
NEG_SLOPE = 0.2
K = 20
N = 1024
B = 4
BN = B * N
NW = 32
PPW = BN // NW
PC = 8
BLK = 128
HI = lax.Precision.HIGHEST

LAYERS = [
    (1, 16 // 3 + 1, 21, 32),
    (21, 32, 21, 32),
    (21, 32, 42, 48),
    (42, 48, 85, 96),
]
C3P = [16, 96, 96, 144]


def _padk(w_t, cp_in, copm):
    cin, cout = w_t.shape
    wp = jnp.zeros((cp_in, copm), jnp.float32).at[:cin, :cout].set(w_t)
    return jnp.kron(jnp.eye(3, dtype=jnp.float32), wp)


def _pad_row(w_t, c3p, copm):
    cin, cout = w_t.shape
    k3 = jnp.kron(jnp.eye(3, dtype=jnp.float32), w_t)
    out = jnp.zeros((c3p, 3 * copm), jnp.float32)
    col = jnp.zeros((3 * copm,), jnp.float32)
    full = jnp.zeros((3, 3 * copm), jnp.float32)
    for d in range(3):
        full = full.at[:, d * copm:d * copm + cout].set(k3[:, d * cout:(d + 1) * cout])
    del col
    return out.at[:3, :].set(full)


def _tc_knn_kernel(x_ref, idx_ref):
    b = pl.program_id(0)
    x = x_ref[...]
    xb = x.astype(jnp.bfloat16)
    g = lax.dot_general(xb, xb, (((1,), (1,)), ((), ())),
                        preferred_element_type=jnp.float32)
    xx2 = x * x
    sq_col = jnp.sum(xx2, axis=1, keepdims=True)
    ones = jnp.ones((1, x.shape[1]), jnp.float32)
    sq_row = lax.dot_general(ones, xx2, (((1,), (1,)), ((), ())),
                             precision=HI, preferred_element_type=jnp.float32)
    pairwise = (-sq_col - (-2.0 * g)) - sq_row

    ci = lax.broadcasted_iota(jnp.int32, (N, N), 1)
    cik = lax.broadcasted_iota(jnp.int32, (N, K), 1)
    base = b * N
    idxacc = lax.broadcasted_iota(jnp.int32, (N, K), 0) + base
    work = pairwise
    for t in range(K):
        m = jnp.max(work, axis=1, keepdims=True)
        sel = jnp.where(work == m, ci, N)
        idx_t = jnp.min(sel, axis=1, keepdims=True)
        idxacc = jnp.where(cik == t, idx_t + base, idxacc)
        work = jnp.where(ci == idx_t, -jnp.inf, work)
    idx_ref[...] = idxacc


def _tc_knn(x2, c3p):
    return pl.pallas_call(
        _tc_knn_kernel,
        grid=(B,),
        in_specs=[pl.BlockSpec((N, c3p), lambda b: (b, 0))],
        out_specs=pl.BlockSpec((N, K), lambda b: (b, 0)),
        out_shape=jax.ShapeDtypeStruct((BN, K), jnp.int32),
    )(x2)


def _sc_gather_body(idx_hbm, xtab_hbm, out_hbm, idx_v, gbuf0, gbuf1,
                    sem0, sem1):
    cid = lax.axis_index("c")
    sid = lax.axis_index("s")
    wid = sid * 2 + cid
    ebase = wid * (PPW * K)

    pltpu.sync_copy(idx_hbm.at[pl.ds(ebase, PPW * K)], idx_v)

    nchunk = PPW // PC
    ec = PC * K

    pltpu.async_copy(xtab_hbm.at[idx_v.at[pl.ds(0, ec)]], gbuf0, sem0)

    def chunk(i, _):
        nxt = i + 1

        @pl.when(jnp.logical_and(nxt < nchunk, lax.rem(nxt, 2) == 0))
        def _():
            pltpu.async_copy(
                xtab_hbm.at[idx_v.at[pl.ds(nxt * ec, ec)]], gbuf0, sem0)

        @pl.when(jnp.logical_and(nxt < nchunk, lax.rem(nxt, 2) == 1))
        def _():
            pltpu.async_copy(
                xtab_hbm.at[idx_v.at[pl.ds(nxt * ec, ec)]], gbuf1, sem1)

        @pl.when(lax.rem(i, 2) == 0)
        def _():
            pltpu.make_async_copy(
                xtab_hbm.at[idx_v.at[pl.ds(i * ec, ec)]], gbuf0, sem0).wait()
            pltpu.sync_copy(gbuf0, out_hbm.at[pl.ds(ebase + i * ec, ec)])

        @pl.when(lax.rem(i, 2) == 1)
        def _():
            pltpu.make_async_copy(
                xtab_hbm.at[idx_v.at[pl.ds(i * ec, ec)]], gbuf1, sem1).wait()
            pltpu.sync_copy(gbuf1, out_hbm.at[pl.ds(ebase + i * ec, ec)])

        return ()

    lax.fori_loop(0, nchunk, chunk, ())


def _sc_gather(idx, xtab, c3p):
    mesh = plsc.VectorSubcoreMesh(core_axis_name="c", subcore_axis_name="s")
    kern = pl.kernel(
        _sc_gather_body,
        out_type=jax.ShapeDtypeStruct((BN * K, c3p), jnp.float32),
        mesh=mesh,
        compiler_params=pltpu.CompilerParams(use_tc_tiling_on_sc=False),
        scratch_types=[
            pltpu.VMEM((PPW * K,), jnp.int32),
            pltpu.VMEM((PC * K, c3p), jnp.float32),
            pltpu.VMEM((PC * K, c3p), jnp.float32),
            pltpu.SemaphoreType.DMA,
            pltpu.SemaphoreType.DMA,
        ],
    )
    return kern(idx.reshape(BN * K), xtab)


CP = 128


def _tc_conv_kernel(coutp, g_ref, x_ref, wd_ref, wc_ref, out_ref):
    gath = g_ref[...]
    xc = x_ref[...]
    c3p = xc.shape[1]
    g3 = gath.reshape(BLK, K, c3p)
    diff = g3 - xc[:, None, :]
    diffb = diff.reshape(BLK * K, c3p).astype(jnp.bfloat16)
    xb = xc.astype(jnp.bfloat16)
    pd1 = jnp.dot(diffb, wd_ref[...], preferred_element_type=jnp.float32)
    pd2 = jnp.dot(xb, wc_ref[...], preferred_element_type=jnp.float32)
    h = pd1.reshape(BLK, K, 6 * CP) + pd2[:, None, :]
    px, py, pz = h[:, :, 0:CP], h[:, :, CP:2 * CP], h[:, :, 2 * CP:3 * CP]
    dx, dy, dz = (h[:, :, 3 * CP:4 * CP], h[:, :, 4 * CP:5 * CP],
                  h[:, :, 5 * CP:6 * CP])
    dot = px * dx + py * dy + pz * dz
    dnsq = dx * dx + dy * dy + dz * dz
    mask = (dot >= 0).astype(jnp.float32)
    coef = (1.0 - mask) * (dot / (dnsq + EPS))
    ons = 1.0 - NEG_SLOPE
    ox = NEG_SLOPE * px + ons * (px - coef * dx)
    oy = NEG_SLOPE * py + ons * (py - coef * dy)
    oz = NEG_SLOPE * pz + ons * (pz - coef * dz)
    out_ref[...] = jnp.concatenate(
        [jnp.sum(ox, axis=1)[:, :coutp] / K,
         jnp.sum(oy, axis=1)[:, :coutp] / K,
         jnp.sum(oz, axis=1)[:, :coutp] / K],
        axis=1)


def _tc_conv(gmat, x2, wd_b, wc_b, coutp):
    c3p = x2.shape[1]
    w6 = 6 * CP
    nblk = BN // BLK
    return pl.pallas_call(
        functools.partial(_tc_conv_kernel, coutp),
        grid=(nblk,),
        in_specs=[
            pl.BlockSpec((BLK * K, c3p), lambda i: (i, 0)),
            pl.BlockSpec((BLK, c3p), lambda i: (i, 0)),
            pl.BlockSpec((c3p, w6), lambda i: (0, 0)),
            pl.BlockSpec((c3p, w6), lambda i: (0, 0)),
        ],
        out_specs=pl.BlockSpec((BLK, 3 * coutp), lambda i: (i, 0)),
        out_shape=jax.ShapeDtypeStruct((BN, 3 * coutp), jnp.float32),
    )(gmat, x2, wd_b, wc_b)


def _tc_final_kernel(x0_ref, x1_ref, x2_ref, x3_ref,
                     wf0_ref, wf1_ref, wf2_ref, wf3_ref,
                     wd0_ref, wd1_ref, wd2_ref, wd3_ref, out_ref):
    cout = 341
    p = jnp.dot(x0_ref[...].astype(jnp.bfloat16), wf0_ref[...], preferred_element_type=jnp.float32)
    p += jnp.dot(x1_ref[...].astype(jnp.bfloat16), wf1_ref[...], preferred_element_type=jnp.float32)
    p += jnp.dot(x2_ref[...].astype(jnp.bfloat16), wf2_ref[...], preferred_element_type=jnp.float32)
    p += jnp.dot(x3_ref[...].astype(jnp.bfloat16), wf3_ref[...], preferred_element_type=jnp.float32)
    dv = jnp.dot(x0_ref[...].astype(jnp.bfloat16), wd0_ref[...], preferred_element_type=jnp.float32)
    dv += jnp.dot(x1_ref[...].astype(jnp.bfloat16), wd1_ref[...], preferred_element_type=jnp.float32)
    dv += jnp.dot(x2_ref[...].astype(jnp.bfloat16), wd2_ref[...], preferred_element_type=jnp.float32)
    dv += jnp.dot(x3_ref[...].astype(jnp.bfloat16), wd3_ref[...], preferred_element_type=jnp.float32)
    px, py, pz = p[:, 0:cout], p[:, cout:2 * cout], p[:, 2 * cout:3 * cout]
    dx, dy, dz = dv[:, 0:1], dv[:, 1:2], dv[:, 2:3]
    dot = px * dx + py * dy + pz * dz
    dnsq = dx * dx + dy * dy + dz * dz
    mask = (dot >= 0).astype(jnp.float32)
    coef = (1.0 - mask) * (dot / (dnsq + EPS))
    ons = 1.0 - NEG_SLOPE
    ox = NEG_SLOPE * px + ons * (px - coef * dx)
    oy = NEG_SLOPE * py + ons * (py - coef * dy)
    oz = NEG_SLOPE * pz + ons * (pz - coef * dz)
    out_ref[0] = jnp.concatenate(
        [jnp.sum(ox, axis=0, keepdims=True) / N,
         jnp.sum(oy, axis=0, keepdims=True) / N,
         jnp.sum(oz, axis=0, keepdims=True) / N], axis=0)


def _tc_final(xs, wf_list, wd_list):
    in_specs = []
    args = []
    for x in xs:
        f = x.shape[1]
        in_specs.append(pl.BlockSpec((N, f), lambda b: (b, 0)))
        args.append(x)
    for w in wf_list + wd_list:
        in_specs.append(pl.BlockSpec(w.shape, lambda b: (0, 0)))
        args.append(w)
    return pl.pallas_call(
        _tc_final_kernel,
        grid=(B,),
        in_specs=in_specs,
        out_specs=pl.BlockSpec((1, 3, 341), lambda b: (b, 0, 0)),
        out_shape=jax.ShapeDtypeStruct((B, 3, 341), jnp.float32),
    )(*args)


def kernel(x, w_feat_0, w_dir_0, w_feat_1, w_dir_1, w_feat_2, w_dir_2,
           w_feat_3, w_dir_3, w_feat_4, w_dir_4):
    wfs = [w_feat_0, w_feat_1, w_feat_2, w_feat_3]
    wds = [w_dir_0, w_dir_1, w_dir_2, w_dir_3]

    x2 = jnp.transpose(x, (0, 2, 1)).reshape(BN, 3)
    x2 = jnp.pad(x2, ((0, 0), (0, C3P[0] - 3)))

    xs = []
    for li, (cin, _, cout, coutp) in enumerate(LAYERS):
        c3p = C3P[li]
        wf, wd = wfs[li], wds[li]
        if li == 0:
            wdk = jnp.concatenate(
                [_pad_row(wf[:, :cin].T, c3p, CP),
                 _pad_row(wd[:, :cin].T, c3p, CP)], axis=1)
            wck = jnp.concatenate(
                [_pad_row(wf[:, cin:].T, c3p, CP),
                 _pad_row(wd[:, cin:].T, c3p, CP)], axis=1)
        else:
            cp_in = c3p // 3
            wdk = jnp.concatenate(
                [_padk(wf[:, :cin].T, cp_in, CP),
                 _padk(wd[:, :cin].T, cp_in, CP)], axis=1)
            wck = jnp.concatenate(
                [_padk(wf[:, cin:].T, cp_in, CP),
                 _padk(wd[:, cin:].T, cp_in, CP)], axis=1)
        idx = _tc_knn(x2, c3p)
        gmat = _sc_gather(idx, x2, c3p)
        x2 = _tc_conv(gmat, x2, wdk.astype(jnp.bfloat16),
                      wck.astype(jnp.bfloat16), coutp)
        xs.append(x2)

    wf4_list, wd4_list = [], []
    off = 0
    for (cin, _, cout, coutp) in LAYERS:
        wf4_list.append(_padk(w_feat_4[:, off:off + cout].T, coutp, 341).astype(jnp.bfloat16))
        wd4_list.append(_padk(w_dir_4[:, off:off + cout].T, coutp, 1).astype(jnp.bfloat16))
        off += cout
    out = _tc_final(xs, wf4_list, wd4_list)
    return jnp.transpose(out, (0, 2, 1))


# speedup vs baseline: 5.4187x; 1.0686x over previous
import functools

import jax
import jax.numpy as jnp
from jax import lax
from jax.experimental import pallas as pl
from jax.experimental.pallas import tpu as pltpu
from jax.experimental.pallas import tpu_sc as plsc

EPS = 1e-6
NEG_SLOPE = 0.2
K = 20
N = 1024
B = 4
BN = B * N
NW = 32
PPW = BN // NW
PC = 8
BLK = 128
HI = lax.Precision.HIGHEST

LAYERS = [
    (1, 16 // 3 + 1, 21, 32),
    (21, 32, 21, 32),
    (21, 32, 42, 48),
    (42, 48, 85, 96),
]
C3P = [16, 96, 96, 144]


def _padk(w_t, cp_in, copm):
    cin, cout = w_t.shape
    wp = jnp.zeros((cp_in, copm), jnp.float32).at[:cin, :cout].set(w_t)
    return jnp.kron(jnp.eye(3, dtype=jnp.float32), wp)


def _pad_row(w_t, c3p, copm):
    cin, cout = w_t.shape
    k3 = jnp.kron(jnp.eye(3, dtype=jnp.float32), w_t)
    out = jnp.zeros((c3p, 3 * copm), jnp.float32)
    col = jnp.zeros((3 * copm,), jnp.float32)
    full = jnp.zeros((3, 3 * copm), jnp.float32)
    for d in range(3):
        full = full.at[:, d * copm:d * copm + cout].set(k3[:, d * cout:(d + 1) * cout])
    del col
    return out.at[:3, :].set(full)


def _tc_knn_kernel(x_ref, idx_ref):
    b = pl.program_id(0)
    x = x_ref[...]
    xb = x.astype(jnp.bfloat16)
    g = lax.dot_general(xb, xb, (((1,), (1,)), ((), ())),
                        preferred_element_type=jnp.float32)
    xx2 = x * x
    sq_col = jnp.sum(xx2, axis=1, keepdims=True)
    ones = jnp.ones((1, x.shape[1]), jnp.float32)
    sq_row = lax.dot_general(ones, xx2, (((1,), (1,)), ((), ())),
                             precision=HI, preferred_element_type=jnp.float32)
    pairwise = (-sq_col - (-2.0 * g)) - sq_row

    cf = lax.broadcasted_iota(jnp.int32, (N, N), 1).astype(jnp.float32)
    cik = lax.broadcasted_iota(jnp.int32, (N, K), 1).astype(jnp.float32)
    base = b * N
    idxacc = lax.broadcasted_iota(jnp.int32, (N, K), 0).astype(jnp.float32)
    work = pairwise
    for t in range(K):
        m = jnp.max(work, axis=1, keepdims=True)
        sel = jnp.where(work == m, cf, jnp.float32(N))
        idx_t = jnp.min(sel, axis=1, keepdims=True)
        idxacc = jnp.where(cik == t, idx_t, idxacc)
        work = jnp.where(cf == idx_t, -jnp.inf, work)
    idx_ref[...] = idxacc.astype(jnp.int32) + base


def _tc_knn(x2, c3p):
    return pl.pallas_call(
        _tc_knn_kernel,
        grid=(B,),
        in_specs=[pl.BlockSpec((N, c3p), lambda b: (b, 0))],
        out_specs=pl.BlockSpec((N, K), lambda b: (b, 0)),
        out_shape=jax.ShapeDtypeStruct((BN, K), jnp.int32),
    )(x2)


def _sc_gather_body(idx_hbm, xtab_hbm, out_hbm, idx_v, gbuf0, gbuf1,
                    sem0, sem1):
    cid = lax.axis_index("c")
    sid = lax.axis_index("s")
    wid = sid * 2 + cid
    ebase = wid * (PPW * K)

    pltpu.sync_copy(idx_hbm.at[pl.ds(ebase, PPW * K)], idx_v)

    nchunk = PPW // PC
    ec = PC * K

    pltpu.async_copy(xtab_hbm.at[idx_v.at[pl.ds(0, ec)]], gbuf0, sem0)

    def chunk(i, _):
        nxt = i + 1

        @pl.when(jnp.logical_and(nxt < nchunk, lax.rem(nxt, 2) == 0))
        def _():
            pltpu.async_copy(
                xtab_hbm.at[idx_v.at[pl.ds(nxt * ec, ec)]], gbuf0, sem0)

        @pl.when(jnp.logical_and(nxt < nchunk, lax.rem(nxt, 2) == 1))
        def _():
            pltpu.async_copy(
                xtab_hbm.at[idx_v.at[pl.ds(nxt * ec, ec)]], gbuf1, sem1)

        @pl.when(lax.rem(i, 2) == 0)
        def _():
            pltpu.make_async_copy(
                xtab_hbm.at[idx_v.at[pl.ds(i * ec, ec)]], gbuf0, sem0).wait()
            pltpu.sync_copy(gbuf0, out_hbm.at[pl.ds(ebase + i * ec, ec)])

        @pl.when(lax.rem(i, 2) == 1)
        def _():
            pltpu.make_async_copy(
                xtab_hbm.at[idx_v.at[pl.ds(i * ec, ec)]], gbuf1, sem1).wait()
            pltpu.sync_copy(gbuf1, out_hbm.at[pl.ds(ebase + i * ec, ec)])

        return ()

    lax.fori_loop(0, nchunk, chunk, ())


def _sc_gather(idx, xtab, c3p):
    mesh = plsc.VectorSubcoreMesh(core_axis_name="c", subcore_axis_name="s")
    kern = pl.kernel(
        _sc_gather_body,
        out_type=jax.ShapeDtypeStruct((BN * K, c3p), jnp.float32),
        mesh=mesh,
        compiler_params=pltpu.CompilerParams(use_tc_tiling_on_sc=False),
        scratch_types=[
            pltpu.VMEM((PPW * K,), jnp.int32),
            pltpu.VMEM((PC * K, c3p), jnp.float32),
            pltpu.VMEM((PC * K, c3p), jnp.float32),
            pltpu.SemaphoreType.DMA,
            pltpu.SemaphoreType.DMA,
        ],
    )
    return kern(idx.reshape(BN * K), xtab)


CP = 128


def _tc_conv_kernel(coutp, g_ref, x_ref, wd_ref, wc_ref, out_ref):
    gath = g_ref[...]
    xc = x_ref[...]
    c3p = xc.shape[1]
    g3 = gath.reshape(BLK, K, c3p)
    diff = g3 - xc[:, None, :]
    diffb = diff.reshape(BLK * K, c3p).astype(jnp.bfloat16)
    xb = xc.astype(jnp.bfloat16)
    pd1 = jnp.dot(diffb, wd_ref[...], preferred_element_type=jnp.float32)
    pd2 = jnp.dot(xb, wc_ref[...], preferred_element_type=jnp.float32)
    h = pd1.reshape(BLK, K, 6 * CP) + pd2[:, None, :]
    px, py, pz = h[:, :, 0:CP], h[:, :, CP:2 * CP], h[:, :, 2 * CP:3 * CP]
    dx, dy, dz = (h[:, :, 3 * CP:4 * CP], h[:, :, 4 * CP:5 * CP],
                  h[:, :, 5 * CP:6 * CP])
    dot = px * dx + py * dy + pz * dz
    dnsq = dx * dx + dy * dy + dz * dz
    mask = (dot >= 0).astype(jnp.float32)
    coef = (1.0 - mask) * (dot / (dnsq + EPS))
    ons = 1.0 - NEG_SLOPE
    ox = NEG_SLOPE * px + ons * (px - coef * dx)
    oy = NEG_SLOPE * py + ons * (py - coef * dy)
    oz = NEG_SLOPE * pz + ons * (pz - coef * dz)
    out_ref[...] = jnp.concatenate(
        [jnp.sum(ox, axis=1)[:, :coutp] / K,
         jnp.sum(oy, axis=1)[:, :coutp] / K,
         jnp.sum(oz, axis=1)[:, :coutp] / K],
        axis=1)


def _tc_conv(gmat, x2, wd_b, wc_b, coutp):
    c3p = x2.shape[1]
    w6 = 6 * CP
    nblk = BN // BLK
    return pl.pallas_call(
        functools.partial(_tc_conv_kernel, coutp),
        grid=(nblk,),
        in_specs=[
            pl.BlockSpec((BLK * K, c3p), lambda i: (i, 0)),
            pl.BlockSpec((BLK, c3p), lambda i: (i, 0)),
            pl.BlockSpec((c3p, w6), lambda i: (0, 0)),
            pl.BlockSpec((c3p, w6), lambda i: (0, 0)),
        ],
        out_specs=pl.BlockSpec((BLK, 3 * coutp), lambda i: (i, 0)),
        out_shape=jax.ShapeDtypeStruct((BN, 3 * coutp), jnp.float32),
    )(gmat, x2, wd_b, wc_b)


def _tc_final_kernel(x0_ref, x1_ref, x2_ref, x3_ref,
                     wf0_ref, wf1_ref, wf2_ref, wf3_ref,
                     wd0_ref, wd1_ref, wd2_ref, wd3_ref, out_ref):
    cout = 341
    p = jnp.dot(x0_ref[...].astype(jnp.bfloat16), wf0_ref[...], preferred_element_type=jnp.float32)
    p += jnp.dot(x1_ref[...].astype(jnp.bfloat16), wf1_ref[...], preferred_element_type=jnp.float32)
    p += jnp.dot(x2_ref[...].astype(jnp.bfloat16), wf2_ref[...], preferred_element_type=jnp.float32)
    p += jnp.dot(x3_ref[...].astype(jnp.bfloat16), wf3_ref[...], preferred_element_type=jnp.float32)
    dv = jnp.dot(x0_ref[...].astype(jnp.bfloat16), wd0_ref[...], preferred_element_type=jnp.float32)
    dv += jnp.dot(x1_ref[...].astype(jnp.bfloat16), wd1_ref[...], preferred_element_type=jnp.float32)
    dv += jnp.dot(x2_ref[...].astype(jnp.bfloat16), wd2_ref[...], preferred_element_type=jnp.float32)
    dv += jnp.dot(x3_ref[...].astype(jnp.bfloat16), wd3_ref[...], preferred_element_type=jnp.float32)
    px, py, pz = p[:, 0:cout], p[:, cout:2 * cout], p[:, 2 * cout:3 * cout]
    dx, dy, dz = dv[:, 0:1], dv[:, 1:2], dv[:, 2:3]
    dot = px * dx + py * dy + pz * dz
    dnsq = dx * dx + dy * dy + dz * dz
    mask = (dot >= 0).astype(jnp.float32)
    coef = (1.0 - mask) * (dot / (dnsq + EPS))
    ons = 1.0 - NEG_SLOPE
    ox = NEG_SLOPE * px + ons * (px - coef * dx)
    oy = NEG_SLOPE * py + ons * (py - coef * dy)
    oz = NEG_SLOPE * pz + ons * (pz - coef * dz)
    out_ref[0] = jnp.concatenate(
        [jnp.sum(ox, axis=0, keepdims=True) / N,
         jnp.sum(oy, axis=0, keepdims=True) / N,
         jnp.sum(oz, axis=0, keepdims=True) / N], axis=0)


def _tc_final(xs, wf_list, wd_list):
    in_specs = []
    args = []
    for x in xs:
        f = x.shape[1]
        in_specs.append(pl.BlockSpec((N, f), lambda b: (b, 0)))
        args.append(x)
    for w in wf_list + wd_list:
        in_specs.append(pl.BlockSpec(w.shape, lambda b: (0, 0)))
        args.append(w)
    return pl.pallas_call(
        _tc_final_kernel,
        grid=(B,),
        in_specs=in_specs,
        out_specs=pl.BlockSpec((1, 3, 341), lambda b: (b, 0, 0)),
        out_shape=jax.ShapeDtypeStruct((B, 3, 341), jnp.float32),
    )(*args)


def kernel(x, w_feat_0, w_dir_0, w_feat_1, w_dir_1, w_feat_2, w_dir_2,
           w_feat_3, w_dir_3, w_feat_4, w_dir_4):
    wfs = [w_feat_0, w_feat_1, w_feat_2, w_feat_3]
    wds = [w_dir_0, w_dir_1, w_dir_2, w_dir_3]

    x2 = jnp.transpose(x, (0, 2, 1)).reshape(BN, 3)
    x2 = jnp.pad(x2, ((0, 0), (0, C3P[0] - 3)))

    xs = []
    for li, (cin, _, cout, coutp) in enumerate(LAYERS):
        c3p = C3P[li]
        wf, wd = wfs[li], wds[li]
        if li == 0:
            wdk = jnp.concatenate(
                [_pad_row(wf[:, :cin].T, c3p, CP),
                 _pad_row(wd[:, :cin].T, c3p, CP)], axis=1)
            wck = jnp.concatenate(
                [_pad_row(wf[:, cin:].T, c3p, CP),
                 _pad_row(wd[:, cin:].T, c3p, CP)], axis=1)
        else:
            cp_in = c3p // 3
            wdk = jnp.concatenate(
                [_padk(wf[:, :cin].T, cp_in, CP),
                 _padk(wd[:, :cin].T, cp_in, CP)], axis=1)
            wck = jnp.concatenate(
                [_padk(wf[:, cin:].T, cp_in, CP),
                 _padk(wd[:, cin:].T, cp_in, CP)], axis=1)
        idx = _tc_knn(x2, c3p)
        gmat = _sc_gather(idx, x2, c3p)
        x2 = _tc_conv(gmat, x2, wdk.astype(jnp.bfloat16),
                      wck.astype(jnp.bfloat16), coutp)
        xs.append(x2)

    wf4_list, wd4_list = [], []
    off = 0
    for (cin, _, cout, coutp) in LAYERS:
        wf4_list.append(_padk(w_feat_4[:, off:off + cout].T, coutp, 341).astype(jnp.bfloat16))
        wd4_list.append(_padk(w_dir_4[:, off:off + cout].T, coutp, 1).astype(jnp.bfloat16))
        off += cout
    out = _tc_final(xs, wf4_list, wd4_list)
    return jnp.transpose(out, (0, 2, 1))


# speedup vs baseline: 5.4494x; 1.0057x over previous
import functools

import jax
import jax.numpy as jnp
from jax import lax
from jax.experimental import pallas as pl
from jax.experimental.pallas import tpu as pltpu
from jax.experimental.pallas import tpu_sc as plsc

EPS = 1e-6
NEG_SLOPE = 0.2
K = 20
N = 1024
B = 4
BN = B * N
NW = 32
PPW = BN // NW
PC = 8
BLK = 256
HI = lax.Precision.HIGHEST

LAYERS = [
    (1, 16 // 3 + 1, 21, 32),
    (21, 32, 21, 32),
    (21, 32, 42, 48),
    (42, 48, 85, 96),
]
C3P = [16, 96, 96, 144]


def _padk(w_t, cp_in, copm):
    cin, cout = w_t.shape
    wp = jnp.zeros((cp_in, copm), jnp.float32).at[:cin, :cout].set(w_t)
    return jnp.kron(jnp.eye(3, dtype=jnp.float32), wp)


def _pad_row(w_t, c3p, copm):
    cin, cout = w_t.shape
    k3 = jnp.kron(jnp.eye(3, dtype=jnp.float32), w_t)
    out = jnp.zeros((c3p, 3 * copm), jnp.float32)
    col = jnp.zeros((3 * copm,), jnp.float32)
    full = jnp.zeros((3, 3 * copm), jnp.float32)
    for d in range(3):
        full = full.at[:, d * copm:d * copm + cout].set(k3[:, d * cout:(d + 1) * cout])
    del col
    return out.at[:3, :].set(full)


def _tc_knn_kernel(x_ref, idx_ref):
    b = pl.program_id(0)
    x = x_ref[...]
    xb = x.astype(jnp.bfloat16)
    g = lax.dot_general(xb, xb, (((1,), (1,)), ((), ())),
                        preferred_element_type=jnp.float32)
    xx2 = x * x
    sq_col = jnp.sum(xx2, axis=1, keepdims=True)
    ones = jnp.ones((1, x.shape[1]), jnp.float32)
    sq_row = lax.dot_general(ones, xx2, (((1,), (1,)), ((), ())),
                             precision=HI, preferred_element_type=jnp.float32)
    pairwise = (-sq_col - (-2.0 * g)) - sq_row

    cf = lax.broadcasted_iota(jnp.int32, (N, N), 1).astype(jnp.float32)
    cik = lax.broadcasted_iota(jnp.int32, (N, K), 1).astype(jnp.float32)
    base = b * N
    idxacc = lax.broadcasted_iota(jnp.int32, (N, K), 0).astype(jnp.float32)
    work = pairwise
    for t in range(K):
        m = jnp.max(work, axis=1, keepdims=True)
        sel = jnp.where(work == m, cf, jnp.float32(N))
        idx_t = jnp.min(sel, axis=1, keepdims=True)
        idxacc = jnp.where(cik == t, idx_t, idxacc)
        work = jnp.where(cf == idx_t, -jnp.inf, work)
    idx_ref[...] = idxacc.astype(jnp.int32) + base


def _tc_knn(x2, c3p):
    return pl.pallas_call(
        _tc_knn_kernel,
        grid=(B,),
        in_specs=[pl.BlockSpec((N, c3p), lambda b: (b, 0))],
        out_specs=pl.BlockSpec((N, K), lambda b: (b, 0)),
        out_shape=jax.ShapeDtypeStruct((BN, K), jnp.int32),
    )(x2)


def _sc_gather_body(idx_hbm, xtab_hbm, out_hbm, idx_v, gbuf0, gbuf1,
                    sem0, sem1):
    cid = lax.axis_index("c")
    sid = lax.axis_index("s")
    wid = sid * 2 + cid
    ebase = wid * (PPW * K)

    pltpu.sync_copy(idx_hbm.at[pl.ds(ebase, PPW * K)], idx_v)

    nchunk = PPW // PC
    ec = PC * K

    pltpu.async_copy(xtab_hbm.at[idx_v.at[pl.ds(0, ec)]], gbuf0, sem0)

    def chunk(i, _):
        nxt = i + 1

        @pl.when(jnp.logical_and(nxt < nchunk, lax.rem(nxt, 2) == 0))
        def _():
            pltpu.async_copy(
                xtab_hbm.at[idx_v.at[pl.ds(nxt * ec, ec)]], gbuf0, sem0)

        @pl.when(jnp.logical_and(nxt < nchunk, lax.rem(nxt, 2) == 1))
        def _():
            pltpu.async_copy(
                xtab_hbm.at[idx_v.at[pl.ds(nxt * ec, ec)]], gbuf1, sem1)

        @pl.when(lax.rem(i, 2) == 0)
        def _():
            pltpu.make_async_copy(
                xtab_hbm.at[idx_v.at[pl.ds(i * ec, ec)]], gbuf0, sem0).wait()
            pltpu.sync_copy(gbuf0, out_hbm.at[pl.ds(ebase + i * ec, ec)])

        @pl.when(lax.rem(i, 2) == 1)
        def _():
            pltpu.make_async_copy(
                xtab_hbm.at[idx_v.at[pl.ds(i * ec, ec)]], gbuf1, sem1).wait()
            pltpu.sync_copy(gbuf1, out_hbm.at[pl.ds(ebase + i * ec, ec)])

        return ()

    lax.fori_loop(0, nchunk, chunk, ())


def _sc_gather(idx, xtab, c3p):
    mesh = plsc.VectorSubcoreMesh(core_axis_name="c", subcore_axis_name="s")
    kern = pl.kernel(
        _sc_gather_body,
        out_type=jax.ShapeDtypeStruct((BN * K, c3p), jnp.float32),
        mesh=mesh,
        compiler_params=pltpu.CompilerParams(use_tc_tiling_on_sc=False),
        scratch_types=[
            pltpu.VMEM((PPW * K,), jnp.int32),
            pltpu.VMEM((PC * K, c3p), jnp.float32),
            pltpu.VMEM((PC * K, c3p), jnp.float32),
            pltpu.SemaphoreType.DMA,
            pltpu.SemaphoreType.DMA,
        ],
    )
    return kern(idx.reshape(BN * K), xtab)


CP = 128


def _tc_conv_kernel(coutp, g_ref, x_ref, wd_ref, wc_ref, out_ref):
    gath = g_ref[...]
    xc = x_ref[...]
    c3p = xc.shape[1]
    g3 = gath.reshape(BLK, K, c3p)
    diff = g3 - xc[:, None, :]
    diffb = diff.reshape(BLK * K, c3p).astype(jnp.bfloat16)
    xb = xc.astype(jnp.bfloat16)
    pd1 = jnp.dot(diffb, wd_ref[...], preferred_element_type=jnp.float32)
    pd2 = jnp.dot(xb, wc_ref[...], preferred_element_type=jnp.float32)
    h = pd1.reshape(BLK, K, 6 * CP) + pd2[:, None, :]
    px, py, pz = h[:, :, 0:CP], h[:, :, CP:2 * CP], h[:, :, 2 * CP:3 * CP]
    dx, dy, dz = (h[:, :, 3 * CP:4 * CP], h[:, :, 4 * CP:5 * CP],
                  h[:, :, 5 * CP:6 * CP])
    dot = px * dx + py * dy + pz * dz
    dnsq = dx * dx + dy * dy + dz * dz
    mask = (dot >= 0).astype(jnp.float32)
    coef = (1.0 - mask) * (dot / (dnsq + EPS))
    ons = 1.0 - NEG_SLOPE
    ox = NEG_SLOPE * px + ons * (px - coef * dx)
    oy = NEG_SLOPE * py + ons * (py - coef * dy)
    oz = NEG_SLOPE * pz + ons * (pz - coef * dz)
    out_ref[...] = jnp.concatenate(
        [jnp.sum(ox, axis=1)[:, :coutp] / K,
         jnp.sum(oy, axis=1)[:, :coutp] / K,
         jnp.sum(oz, axis=1)[:, :coutp] / K],
        axis=1)


def _tc_conv(gmat, x2, wd_b, wc_b, coutp):
    c3p = x2.shape[1]
    w6 = 6 * CP
    nblk = BN // BLK
    return pl.pallas_call(
        functools.partial(_tc_conv_kernel, coutp),
        grid=(nblk,),
        in_specs=[
            pl.BlockSpec((BLK * K, c3p), lambda i: (i, 0)),
            pl.BlockSpec((BLK, c3p), lambda i: (i, 0)),
            pl.BlockSpec((c3p, w6), lambda i: (0, 0)),
            pl.BlockSpec((c3p, w6), lambda i: (0, 0)),
        ],
        out_specs=pl.BlockSpec((BLK, 3 * coutp), lambda i: (i, 0)),
        out_shape=jax.ShapeDtypeStruct((BN, 3 * coutp), jnp.float32),
    )(gmat, x2, wd_b, wc_b)


def _tc_final_kernel(x0_ref, x1_ref, x2_ref, x3_ref,
                     wf0_ref, wf1_ref, wf2_ref, wf3_ref,
                     wd0_ref, wd1_ref, wd2_ref, wd3_ref, out_ref):
    cout = 341
    p = jnp.dot(x0_ref[...].astype(jnp.bfloat16), wf0_ref[...], preferred_element_type=jnp.float32)
    p += jnp.dot(x1_ref[...].astype(jnp.bfloat16), wf1_ref[...], preferred_element_type=jnp.float32)
    p += jnp.dot(x2_ref[...].astype(jnp.bfloat16), wf2_ref[...], preferred_element_type=jnp.float32)
    p += jnp.dot(x3_ref[...].astype(jnp.bfloat16), wf3_ref[...], preferred_element_type=jnp.float32)
    dv = jnp.dot(x0_ref[...].astype(jnp.bfloat16), wd0_ref[...], preferred_element_type=jnp.float32)
    dv += jnp.dot(x1_ref[...].astype(jnp.bfloat16), wd1_ref[...], preferred_element_type=jnp.float32)
    dv += jnp.dot(x2_ref[...].astype(jnp.bfloat16), wd2_ref[...], preferred_element_type=jnp.float32)
    dv += jnp.dot(x3_ref[...].astype(jnp.bfloat16), wd3_ref[...], preferred_element_type=jnp.float32)
    px, py, pz = p[:, 0:cout], p[:, cout:2 * cout], p[:, 2 * cout:3 * cout]
    dx, dy, dz = dv[:, 0:1], dv[:, 1:2], dv[:, 2:3]
    dot = px * dx + py * dy + pz * dz
    dnsq = dx * dx + dy * dy + dz * dz
    mask = (dot >= 0).astype(jnp.float32)
    coef = (1.0 - mask) * (dot / (dnsq + EPS))
    ons = 1.0 - NEG_SLOPE
    ox = NEG_SLOPE * px + ons * (px - coef * dx)
    oy = NEG_SLOPE * py + ons * (py - coef * dy)
    oz = NEG_SLOPE * pz + ons * (pz - coef * dz)
    out_ref[0] = jnp.concatenate(
        [jnp.sum(ox, axis=0, keepdims=True) / N,
         jnp.sum(oy, axis=0, keepdims=True) / N,
         jnp.sum(oz, axis=0, keepdims=True) / N], axis=0)


def _tc_final(xs, wf_list, wd_list):
    in_specs = []
    args = []
    for x in xs:
        f = x.shape[1]
        in_specs.append(pl.BlockSpec((N, f), lambda b: (b, 0)))
        args.append(x)
    for w in wf_list + wd_list:
        in_specs.append(pl.BlockSpec(w.shape, lambda b: (0, 0)))
        args.append(w)
    return pl.pallas_call(
        _tc_final_kernel,
        grid=(B,),
        in_specs=in_specs,
        out_specs=pl.BlockSpec((1, 3, 341), lambda b: (b, 0, 0)),
        out_shape=jax.ShapeDtypeStruct((B, 3, 341), jnp.float32),
    )(*args)


def kernel(x, w_feat_0, w_dir_0, w_feat_1, w_dir_1, w_feat_2, w_dir_2,
           w_feat_3, w_dir_3, w_feat_4, w_dir_4):
    wfs = [w_feat_0, w_feat_1, w_feat_2, w_feat_3]
    wds = [w_dir_0, w_dir_1, w_dir_2, w_dir_3]

    x2 = jnp.transpose(x, (0, 2, 1)).reshape(BN, 3)
    x2 = jnp.pad(x2, ((0, 0), (0, C3P[0] - 3)))

    xs = []
    for li, (cin, _, cout, coutp) in enumerate(LAYERS):
        c3p = C3P[li]
        wf, wd = wfs[li], wds[li]
        if li == 0:
            wdk = jnp.concatenate(
                [_pad_row(wf[:, :cin].T, c3p, CP),
                 _pad_row(wd[:, :cin].T, c3p, CP)], axis=1)
            wck = jnp.concatenate(
                [_pad_row(wf[:, cin:].T, c3p, CP),
                 _pad_row(wd[:, cin:].T, c3p, CP)], axis=1)
        else:
            cp_in = c3p // 3
            wdk = jnp.concatenate(
                [_padk(wf[:, :cin].T, cp_in, CP),
                 _padk(wd[:, :cin].T, cp_in, CP)], axis=1)
            wck = jnp.concatenate(
                [_padk(wf[:, cin:].T, cp_in, CP),
                 _padk(wd[:, cin:].T, cp_in, CP)], axis=1)
        idx = _tc_knn(x2, c3p)
        gmat = _sc_gather(idx, x2, c3p)
        x2 = _tc_conv(gmat, x2, wdk.astype(jnp.bfloat16),
                      wck.astype(jnp.bfloat16), coutp)
        xs.append(x2)

    wf4_list, wd4_list = [], []
    off = 0
    for (cin, _, cout, coutp) in LAYERS:
        wf4_list.append(_padk(w_feat_4[:, off:off + cout].T, coutp, 341).astype(jnp.bfloat16))
        wd4_list.append(_padk(w_dir_4[:, off:off + cout].T, coutp, 1).astype(jnp.bfloat16))
        off += cout
    out = _tc_final(xs, wf4_list, wd4_list)
    return jnp.transpose(out, (0, 2, 1))


# speedup vs baseline: 6.1875x; 1.1354x over previous
import functools

import jax
import jax.numpy as jnp
from jax import lax
from jax.experimental import pallas as pl
from jax.experimental.pallas import tpu as pltpu
from jax.experimental.pallas import tpu_sc as plsc

EPS = 1e-6
NEG_SLOPE = 0.2
K = 20
N = 1024
B = 4
BN = B * N
NW = 32
PPW = BN // NW
PC = 8
BLK = 256
HI = lax.Precision.HIGHEST

LAYERS = [
    (1, 16 // 3 + 1, 21, 32),
    (21, 32, 21, 32),
    (21, 32, 42, 48),
    (42, 48, 85, 96),
]
C3P = [16, 96, 96, 144]


def _padk(w_t, cp_in, copm):
    cin, cout = w_t.shape
    wp = jnp.zeros((cp_in, copm), jnp.float32).at[:cin, :cout].set(w_t)
    return jnp.kron(jnp.eye(3, dtype=jnp.float32), wp)


def _pad_row(w_t, c3p, copm):
    cin, cout = w_t.shape
    k3 = jnp.kron(jnp.eye(3, dtype=jnp.float32), w_t)
    out = jnp.zeros((c3p, 3 * copm), jnp.float32)
    col = jnp.zeros((3 * copm,), jnp.float32)
    full = jnp.zeros((3, 3 * copm), jnp.float32)
    for d in range(3):
        full = full.at[:, d * copm:d * copm + cout].set(k3[:, d * cout:(d + 1) * cout])
    del col
    return out.at[:3, :].set(full)


def _tc_knn_kernel(x_ref, idx_ref):
    b = pl.program_id(0)
    x = x_ref[...]
    xb = x.astype(jnp.bfloat16)
    g = lax.dot_general(xb, xb, (((1,), (1,)), ((), ())),
                        preferred_element_type=jnp.float32)
    xx2 = x * x
    sq_col = jnp.sum(xx2, axis=1, keepdims=True)
    ones = jnp.ones((1, x.shape[1]), jnp.float32)
    sq_row = lax.dot_general(ones, xx2, (((1,), (1,)), ((), ())),
                             precision=HI, preferred_element_type=jnp.float32)
    pairwise = (-sq_col - (-2.0 * g)) - sq_row

    cf = lax.broadcasted_iota(jnp.int32, (N, N), 1).astype(jnp.float32)
    cik = lax.broadcasted_iota(jnp.int32, (N, K), 1).astype(jnp.float32)
    base = b * N
    idxacc = lax.broadcasted_iota(jnp.int32, (N, K), 0).astype(jnp.float32)
    work = pairwise
    for t in range(K):
        m = jnp.max(work, axis=1, keepdims=True)
        sel = jnp.where(work == m, cf, jnp.float32(N))
        idx_t = jnp.min(sel, axis=1, keepdims=True)
        idxacc = jnp.where(cik == t, idx_t, idxacc)
        work = jnp.where(cf == idx_t, -jnp.inf, work)
    idx_ref[...] = idxacc.astype(jnp.int32) + base


def _tc_knn(x2, c3p):
    return pl.pallas_call(
        _tc_knn_kernel,
        grid=(B,),
        in_specs=[pl.BlockSpec((N, c3p), lambda b: (b, 0))],
        out_specs=pl.BlockSpec((N, K), lambda b: (b, 0)),
        out_shape=jax.ShapeDtypeStruct((BN, K), jnp.int32),
    )(x2)


def _sc_gather_body(idx_hbm, xtab_hbm, out_hbm, idx_v, gbuf0, gbuf1,
                    sem0, sem1):
    cid = lax.axis_index("c")
    sid = lax.axis_index("s")
    wid = sid * 2 + cid
    ebase = wid * (PPW * K)

    pltpu.sync_copy(idx_hbm.at[pl.ds(ebase, PPW * K)], idx_v)

    nchunk = PPW // PC
    ec = PC * K

    pltpu.async_copy(xtab_hbm.at[idx_v.at[pl.ds(0, ec)]], gbuf0, sem0)

    def chunk(i, _):
        nxt = i + 1

        @pl.when(jnp.logical_and(nxt < nchunk, lax.rem(nxt, 2) == 0))
        def _():
            pltpu.async_copy(
                xtab_hbm.at[idx_v.at[pl.ds(nxt * ec, ec)]], gbuf0, sem0)

        @pl.when(jnp.logical_and(nxt < nchunk, lax.rem(nxt, 2) == 1))
        def _():
            pltpu.async_copy(
                xtab_hbm.at[idx_v.at[pl.ds(nxt * ec, ec)]], gbuf1, sem1)

        @pl.when(lax.rem(i, 2) == 0)
        def _():
            pltpu.make_async_copy(
                xtab_hbm.at[idx_v.at[pl.ds(i * ec, ec)]], gbuf0, sem0).wait()
            pltpu.sync_copy(gbuf0, out_hbm.at[pl.ds(ebase + i * ec, ec)])

        @pl.when(lax.rem(i, 2) == 1)
        def _():
            pltpu.make_async_copy(
                xtab_hbm.at[idx_v.at[pl.ds(i * ec, ec)]], gbuf1, sem1).wait()
            pltpu.sync_copy(gbuf1, out_hbm.at[pl.ds(ebase + i * ec, ec)])

        return ()

    lax.fori_loop(0, nchunk, chunk, ())


def _sc_gather(idx, xtab, c3p):
    mesh = plsc.VectorSubcoreMesh(core_axis_name="c", subcore_axis_name="s")
    kern = pl.kernel(
        _sc_gather_body,
        out_type=jax.ShapeDtypeStruct((BN * K, c3p), jnp.float32),
        mesh=mesh,
        compiler_params=pltpu.CompilerParams(use_tc_tiling_on_sc=False),
        scratch_types=[
            pltpu.VMEM((PPW * K,), jnp.int32),
            pltpu.VMEM((PC * K, c3p), jnp.float32),
            pltpu.VMEM((PC * K, c3p), jnp.float32),
            pltpu.SemaphoreType.DMA,
            pltpu.SemaphoreType.DMA,
        ],
    )
    return kern(idx.reshape(BN * K), xtab)


CP = 128


def _tc_conv_kernel(coutp, g_ref, x_ref, wd_ref, wc_ref, out_ref):
    gath = g_ref[...]
    xc = x_ref[...]
    c3p = xc.shape[1]
    g3 = gath.reshape(BLK, K, c3p)
    diff = g3 - xc[:, None, :]
    diffb = diff.reshape(BLK * K, c3p).astype(jnp.bfloat16)
    xb = xc.astype(jnp.bfloat16)
    pd1 = jnp.dot(diffb, wd_ref[...], preferred_element_type=jnp.float32)
    pd2 = jnp.dot(xb, wc_ref[...], preferred_element_type=jnp.float32)
    h = pd1.reshape(BLK, K, 6 * CP) + pd2[:, None, :]
    px, py, pz = h[:, :, 0:CP], h[:, :, CP:2 * CP], h[:, :, 2 * CP:3 * CP]
    dx, dy, dz = (h[:, :, 3 * CP:4 * CP], h[:, :, 4 * CP:5 * CP],
                  h[:, :, 5 * CP:6 * CP])
    dot = px * dx + py * dy + pz * dz
    dnsq = dx * dx + dy * dy + dz * dz
    coef = jnp.where(dot >= 0, 0.0, (1.0 - NEG_SLOPE) * (dot / (dnsq + EPS)))
    ox = px - coef * dx
    oy = py - coef * dy
    oz = pz - coef * dz
    out_ref[...] = jnp.concatenate(
        [jnp.sum(ox, axis=1)[:, :coutp] / K,
         jnp.sum(oy, axis=1)[:, :coutp] / K,
         jnp.sum(oz, axis=1)[:, :coutp] / K],
        axis=1)


def _tc_conv(gmat, x2, wd_b, wc_b, coutp):
    c3p = x2.shape[1]
    w6 = 6 * CP
    nblk = BN // BLK
    return pl.pallas_call(
        functools.partial(_tc_conv_kernel, coutp),
        grid=(nblk,),
        in_specs=[
            pl.BlockSpec((BLK * K, c3p), lambda i: (i, 0)),
            pl.BlockSpec((BLK, c3p), lambda i: (i, 0)),
            pl.BlockSpec((c3p, w6), lambda i: (0, 0)),
            pl.BlockSpec((c3p, w6), lambda i: (0, 0)),
        ],
        out_specs=pl.BlockSpec((BLK, 3 * coutp), lambda i: (i, 0)),
        out_shape=jax.ShapeDtypeStruct((BN, 3 * coutp), jnp.float32),
    )(gmat, x2, wd_b, wc_b)


def _tc_final_kernel(x0_ref, x1_ref, x2_ref, x3_ref,
                     wf0_ref, wf1_ref, wf2_ref, wf3_ref,
                     wd0_ref, wd1_ref, wd2_ref, wd3_ref, out_ref):
    cout = 341
    p = jnp.dot(x0_ref[...].astype(jnp.bfloat16), wf0_ref[...], preferred_element_type=jnp.float32)
    p += jnp.dot(x1_ref[...].astype(jnp.bfloat16), wf1_ref[...], preferred_element_type=jnp.float32)
    p += jnp.dot(x2_ref[...].astype(jnp.bfloat16), wf2_ref[...], preferred_element_type=jnp.float32)
    p += jnp.dot(x3_ref[...].astype(jnp.bfloat16), wf3_ref[...], preferred_element_type=jnp.float32)
    dv = jnp.dot(x0_ref[...].astype(jnp.bfloat16), wd0_ref[...], preferred_element_type=jnp.float32)
    dv += jnp.dot(x1_ref[...].astype(jnp.bfloat16), wd1_ref[...], preferred_element_type=jnp.float32)
    dv += jnp.dot(x2_ref[...].astype(jnp.bfloat16), wd2_ref[...], preferred_element_type=jnp.float32)
    dv += jnp.dot(x3_ref[...].astype(jnp.bfloat16), wd3_ref[...], preferred_element_type=jnp.float32)
    px, py, pz = p[:, 0:cout], p[:, cout:2 * cout], p[:, 2 * cout:3 * cout]
    dx, dy, dz = dv[:, 0:1], dv[:, 1:2], dv[:, 2:3]
    dot = px * dx + py * dy + pz * dz
    dnsq = dx * dx + dy * dy + dz * dz
    coef = jnp.where(dot >= 0, 0.0, (1.0 - NEG_SLOPE) * (dot / (dnsq + EPS)))
    ox = px - coef * dx
    oy = py - coef * dy
    oz = pz - coef * dz
    out_ref[0] = jnp.concatenate(
        [jnp.sum(ox, axis=0, keepdims=True) / N,
         jnp.sum(oy, axis=0, keepdims=True) / N,
         jnp.sum(oz, axis=0, keepdims=True) / N], axis=0)


def _tc_final(xs, wf_list, wd_list):
    in_specs = []
    args = []
    for x in xs:
        f = x.shape[1]
        in_specs.append(pl.BlockSpec((N, f), lambda b: (b, 0)))
        args.append(x)
    for w in wf_list + wd_list:
        in_specs.append(pl.BlockSpec(w.shape, lambda b: (0, 0)))
        args.append(w)
    return pl.pallas_call(
        _tc_final_kernel,
        grid=(B,),
        in_specs=in_specs,
        out_specs=pl.BlockSpec((1, 3, 341), lambda b: (b, 0, 0)),
        out_shape=jax.ShapeDtypeStruct((B, 3, 341), jnp.float32),
    )(*args)


def kernel(x, w_feat_0, w_dir_0, w_feat_1, w_dir_1, w_feat_2, w_dir_2,
           w_feat_3, w_dir_3, w_feat_4, w_dir_4):
    wfs = [w_feat_0, w_feat_1, w_feat_2, w_feat_3]
    wds = [w_dir_0, w_dir_1, w_dir_2, w_dir_3]

    x2 = jnp.transpose(x, (0, 2, 1)).reshape(BN, 3)
    x2 = jnp.pad(x2, ((0, 0), (0, C3P[0] - 3)))

    xs = []
    for li, (cin, _, cout, coutp) in enumerate(LAYERS):
        c3p = C3P[li]
        wf, wd = wfs[li], wds[li]
        if li == 0:
            wdk = jnp.concatenate(
                [_pad_row(wf[:, :cin].T, c3p, CP),
                 _pad_row(wd[:, :cin].T, c3p, CP)], axis=1)
            wck = jnp.concatenate(
                [_pad_row(wf[:, cin:].T, c3p, CP),
                 _pad_row(wd[:, cin:].T, c3p, CP)], axis=1)
        else:
            cp_in = c3p // 3
            wdk = jnp.concatenate(
                [_padk(wf[:, :cin].T, cp_in, CP),
                 _padk(wd[:, :cin].T, cp_in, CP)], axis=1)
            wck = jnp.concatenate(
                [_padk(wf[:, cin:].T, cp_in, CP),
                 _padk(wd[:, cin:].T, cp_in, CP)], axis=1)
        idx = _tc_knn(x2, c3p)
        gmat = _sc_gather(idx, x2, c3p)
        x2 = _tc_conv(gmat, x2, wdk.astype(jnp.bfloat16),
                      wck.astype(jnp.bfloat16), coutp)
        xs.append(x2)

    wf4_list, wd4_list = [], []
    off = 0
    for (cin, _, cout, coutp) in LAYERS:
        wf4_list.append(_padk(w_feat_4[:, off:off + cout].T, coutp, 341).astype(jnp.bfloat16))
        wd4_list.append(_padk(w_dir_4[:, off:off + cout].T, coutp, 1).astype(jnp.bfloat16))
        off += cout
    out = _tc_final(xs, wf4_list, wd4_list)
    return jnp.transpose(out, (0, 2, 1))


# speedup vs baseline: 6.2043x; 1.0027x over previous
import functools

import jax
import jax.numpy as jnp
from jax import lax
from jax.experimental import pallas as pl
from jax.experimental.pallas import tpu as pltpu
from jax.experimental.pallas import tpu_sc as plsc

EPS = 1e-6
NEG_SLOPE = 0.2
K = 20
N = 1024
B = 4
BN = B * N
NW = 32
PPW = BN // NW
PC = 8
BLK = 256
HI = lax.Precision.HIGHEST

LAYERS = [
    (1, 0, 21, 32),
    (21, 32, 21, 32),
    (21, 32, 42, 48),
    (42, 48, 85, 96),
]
C3P = [16, 96, 96, 144]


def _padk(w_t, cp_in, copm):
    cin, cout = w_t.shape
    wp = jnp.zeros((cp_in, copm), jnp.float32).at[:cin, :cout].set(w_t)
    return jnp.kron(jnp.eye(3, dtype=jnp.float32), wp)


def _pad_row(w_t, c3p, copm):
    cin, cout = w_t.shape
    k3 = jnp.kron(jnp.eye(3, dtype=jnp.float32), w_t)
    out = jnp.zeros((c3p, 3 * copm), jnp.float32)
    col = jnp.zeros((3 * copm,), jnp.float32)
    full = jnp.zeros((3, 3 * copm), jnp.float32)
    for d in range(3):
        full = full.at[:, d * copm:d * copm + cout].set(k3[:, d * cout:(d + 1) * cout])
    del col
    return out.at[:3, :].set(full)


def _tc_knn_kernel(x_ref, idx_ref):
    b = pl.program_id(0)
    x = x_ref[...]
    xb = x.astype(jnp.bfloat16)
    g = lax.dot_general(xb, xb, (((1,), (1,)), ((), ())),
                        preferred_element_type=jnp.float32)
    xx2 = x * x
    sq_col = jnp.sum(xx2, axis=1, keepdims=True)
    ones = jnp.ones((1, x.shape[1]), jnp.float32)
    sq_row = lax.dot_general(ones, xx2, (((1,), (1,)), ((), ())),
                             precision=HI, preferred_element_type=jnp.float32)
    pairwise = (-sq_col - (-2.0 * g)) - sq_row

    cf = lax.broadcasted_iota(jnp.int32, (N, N), 1).astype(jnp.float32)
    cik = lax.broadcasted_iota(jnp.int32, (N, K), 1).astype(jnp.float32)
    base = b * N
    idxacc = lax.broadcasted_iota(jnp.int32, (N, K), 0).astype(jnp.float32)
    work = pairwise
    for t in range(K):
        m = jnp.max(work, axis=1, keepdims=True)
        sel = jnp.where(work == m, cf, jnp.float32(N))
        idx_t = jnp.min(sel, axis=1, keepdims=True)
        idxacc = jnp.where(cik == t, idx_t, idxacc)
        work = jnp.where(cf == idx_t, -jnp.inf, work)
    idx_ref[...] = idxacc.astype(jnp.int32) + base


def _tc_knn(x2, c3p):
    return pl.pallas_call(
        _tc_knn_kernel,
        grid=(B,),
        in_specs=[pl.BlockSpec((N, c3p), lambda b: (b, 0))],
        out_specs=pl.BlockSpec((N, K), lambda b: (b, 0)),
        out_shape=jax.ShapeDtypeStruct((BN, K), jnp.int32),
    )(x2)


def _sc_gather_body(idx_hbm, xtab_hbm, out_hbm, idx_v, gbuf0, gbuf1,
                    sem0, sem1):
    cid = lax.axis_index("c")
    sid = lax.axis_index("s")
    wid = sid * 2 + cid
    ebase = wid * (PPW * K)

    pltpu.sync_copy(idx_hbm.at[pl.ds(ebase, PPW * K)], idx_v)

    nchunk = PPW // PC
    ec = PC * K

    pltpu.async_copy(xtab_hbm.at[idx_v.at[pl.ds(0, ec)]], gbuf0, sem0)

    def chunk(i, _):
        nxt = i + 1

        @pl.when(jnp.logical_and(nxt < nchunk, lax.rem(nxt, 2) == 0))
        def _():
            pltpu.async_copy(
                xtab_hbm.at[idx_v.at[pl.ds(nxt * ec, ec)]], gbuf0, sem0)

        @pl.when(jnp.logical_and(nxt < nchunk, lax.rem(nxt, 2) == 1))
        def _():
            pltpu.async_copy(
                xtab_hbm.at[idx_v.at[pl.ds(nxt * ec, ec)]], gbuf1, sem1)

        @pl.when(lax.rem(i, 2) == 0)
        def _():
            pltpu.make_async_copy(
                xtab_hbm.at[idx_v.at[pl.ds(i * ec, ec)]], gbuf0, sem0).wait()
            pltpu.sync_copy(gbuf0, out_hbm.at[pl.ds(ebase + i * ec, ec)])

        @pl.when(lax.rem(i, 2) == 1)
        def _():
            pltpu.make_async_copy(
                xtab_hbm.at[idx_v.at[pl.ds(i * ec, ec)]], gbuf1, sem1).wait()
            pltpu.sync_copy(gbuf1, out_hbm.at[pl.ds(ebase + i * ec, ec)])

        return ()

    lax.fori_loop(0, nchunk, chunk, ())


def _sc_gather(idx, xtab, c3p):
    mesh = plsc.VectorSubcoreMesh(core_axis_name="c", subcore_axis_name="s")
    kern = pl.kernel(
        _sc_gather_body,
        out_type=jax.ShapeDtypeStruct((BN * K, c3p), jnp.float32),
        mesh=mesh,
        compiler_params=pltpu.CompilerParams(use_tc_tiling_on_sc=False),
        scratch_types=[
            pltpu.VMEM((PPW * K,), jnp.int32),
            pltpu.VMEM((PC * K, c3p), jnp.float32),
            pltpu.VMEM((PC * K, c3p), jnp.float32),
            pltpu.SemaphoreType.DMA,
            pltpu.SemaphoreType.DMA,
        ],
    )
    return kern(idx.reshape(BN * K), xtab)


CP = 128


def _tc_conv_kernel(coutp, g_ref, x_ref, wd_ref, wc_ref, out_ref):
    gath = g_ref[...]
    xc = x_ref[...]
    c3p = xc.shape[1]
    g3 = gath.reshape(BLK, K, c3p)
    diff = g3 - xc[:, None, :]
    diffb = diff.reshape(BLK * K, c3p).astype(jnp.bfloat16)
    xb = xc.astype(jnp.bfloat16)
    pd1 = jnp.dot(diffb, wd_ref[...], preferred_element_type=jnp.float32)
    pd2 = jnp.dot(xb, wc_ref[...], preferred_element_type=jnp.float32)
    h = pd1.reshape(BLK, K, 6 * CP) + pd2[:, None, :]
    px, py, pz = h[:, :, 0:CP], h[:, :, CP:2 * CP], h[:, :, 2 * CP:3 * CP]
    dx, dy, dz = (h[:, :, 3 * CP:4 * CP], h[:, :, 4 * CP:5 * CP],
                  h[:, :, 5 * CP:6 * CP])
    dot = px * dx + py * dy + pz * dz
    dnsq = dx * dx + dy * dy + dz * dz
    coef = jnp.where(dot >= 0, 0.0, (1.0 - NEG_SLOPE) * (dot / (dnsq + EPS)))
    ox = px - coef * dx
    oy = py - coef * dy
    oz = pz - coef * dz
    out_ref[...] = jnp.concatenate(
        [jnp.sum(ox, axis=1)[:, :coutp] / K,
         jnp.sum(oy, axis=1)[:, :coutp] / K,
         jnp.sum(oz, axis=1)[:, :coutp] / K],
        axis=1)


def _tc_conv(gmat, x2, wd_b, wc_b, coutp):
    c3p = x2.shape[1]
    w6 = 6 * CP
    nblk = BN // BLK
    return pl.pallas_call(
        functools.partial(_tc_conv_kernel, coutp),
        grid=(nblk,),
        in_specs=[
            pl.BlockSpec((BLK * K, c3p), lambda i: (i, 0)),
            pl.BlockSpec((BLK, c3p), lambda i: (i, 0)),
            pl.BlockSpec((c3p, w6), lambda i: (0, 0)),
            pl.BlockSpec((c3p, w6), lambda i: (0, 0)),
        ],
        out_specs=pl.BlockSpec((BLK, 3 * coutp), lambda i: (i, 0)),
        out_shape=jax.ShapeDtypeStruct((BN, 3 * coutp), jnp.float32),
    )(gmat, x2, wd_b, wc_b)


def _tc_final_kernel(x0_ref, x1_ref, x2_ref, x3_ref,
                     wf0_ref, wf1_ref, wf2_ref, wf3_ref,
                     wd0_ref, wd1_ref, wd2_ref, wd3_ref, out_ref):
    cout = 341
    p = jnp.dot(x0_ref[...].astype(jnp.bfloat16), wf0_ref[...], preferred_element_type=jnp.float32)
    p += jnp.dot(x1_ref[...].astype(jnp.bfloat16), wf1_ref[...], preferred_element_type=jnp.float32)
    p += jnp.dot(x2_ref[...].astype(jnp.bfloat16), wf2_ref[...], preferred_element_type=jnp.float32)
    p += jnp.dot(x3_ref[...].astype(jnp.bfloat16), wf3_ref[...], preferred_element_type=jnp.float32)
    dv = jnp.dot(x0_ref[...].astype(jnp.bfloat16), wd0_ref[...], preferred_element_type=jnp.float32)
    dv += jnp.dot(x1_ref[...].astype(jnp.bfloat16), wd1_ref[...], preferred_element_type=jnp.float32)
    dv += jnp.dot(x2_ref[...].astype(jnp.bfloat16), wd2_ref[...], preferred_element_type=jnp.float32)
    dv += jnp.dot(x3_ref[...].astype(jnp.bfloat16), wd3_ref[...], preferred_element_type=jnp.float32)
    px, py, pz = p[:, 0:cout], p[:, cout:2 * cout], p[:, 2 * cout:3 * cout]
    dx, dy, dz = dv[:, 0:1], dv[:, 1:2], dv[:, 2:3]
    dot = px * dx + py * dy + pz * dz
    dnsq = dx * dx + dy * dy + dz * dz
    coef = jnp.where(dot >= 0, 0.0, (1.0 - NEG_SLOPE) * (dot / (dnsq + EPS)))
    ox = px - coef * dx
    oy = py - coef * dy
    oz = pz - coef * dz
    out_ref[0] = jnp.concatenate(
        [jnp.sum(ox, axis=0, keepdims=True) / N,
         jnp.sum(oy, axis=0, keepdims=True) / N,
         jnp.sum(oz, axis=0, keepdims=True) / N], axis=0)


def _tc_final(xs, wf_list, wd_list):
    in_specs = []
    args = []
    for x in xs:
        f = x.shape[1]
        in_specs.append(pl.BlockSpec((N, f), lambda b: (b, 0)))
        args.append(x)
    for w in wf_list + wd_list:
        in_specs.append(pl.BlockSpec(w.shape, lambda b: (0, 0)))
        args.append(w)
    return pl.pallas_call(
        _tc_final_kernel,
        grid=(B,),
        in_specs=in_specs,
        out_specs=pl.BlockSpec((1, 3, 341), lambda b: (b, 0, 0)),
        out_shape=jax.ShapeDtypeStruct((B, 3, 341), jnp.float32),
    )(*args)


def kernel(x, w_feat_0, w_dir_0, w_feat_1, w_dir_1, w_feat_2, w_dir_2,
           w_feat_3, w_dir_3, w_feat_4, w_dir_4):
    wfs = [w_feat_0, w_feat_1, w_feat_2, w_feat_3]
    wds = [w_dir_0, w_dir_1, w_dir_2, w_dir_3]

    x2 = jnp.transpose(x, (0, 2, 1)).reshape(BN, 3)
    x2 = jnp.pad(x2, ((0, 0), (0, C3P[0] - 3)))

    xs = []
    for li, (cin, _, cout, coutp) in enumerate(LAYERS):
        c3p = C3P[li]
        wf, wd = wfs[li], wds[li]
        if li == 0:
            wdk = jnp.concatenate(
                [_pad_row(wf[:, :cin].T, c3p, CP),
                 _pad_row(wd[:, :cin].T, c3p, CP)], axis=1)
            wck = jnp.concatenate(
                [_pad_row(wf[:, cin:].T, c3p, CP),
                 _pad_row(wd[:, cin:].T, c3p, CP)], axis=1)
        else:
            cp_in = c3p // 3
            wdk = jnp.concatenate(
                [_padk(wf[:, :cin].T, cp_in, CP),
                 _padk(wd[:, :cin].T, cp_in, CP)], axis=1)
            wck = jnp.concatenate(
                [_padk(wf[:, cin:].T, cp_in, CP),
                 _padk(wd[:, cin:].T, cp_in, CP)], axis=1)
        idx = _tc_knn(x2, c3p)
        gmat = _sc_gather(idx, x2, c3p)
        x2 = _tc_conv(gmat, x2, wdk.astype(jnp.bfloat16),
                      wck.astype(jnp.bfloat16), coutp)
        xs.append(x2)

    wf4_list, wd4_list = [], []
    off = 0
    for (cin, _, cout, coutp) in LAYERS:
        wf4_list.append(_padk(w_feat_4[:, off:off + cout].T, coutp, 341).astype(jnp.bfloat16))
        wd4_list.append(_padk(w_dir_4[:, off:off + cout].T, coutp, 1).astype(jnp.bfloat16))
        off += cout
    out = _tc_final(xs, wf4_list, wd4_list)
    return jnp.transpose(out, (0, 2, 1))
